# baseline (speedup 1.0000x reference)
import functools

import jax
import jax.numpy as jnp
from jax import lax
from jax.experimental import pallas as pl
from jax.experimental.pallas import tpu as pltpu
from jax.experimental.pallas import tpu_sc as plsc

N = 10000
D = 128
E = 320000
NC = 2
NS = 16
NW = NC * NS
EPW = E // NW
WB = 128
NWIN = EPW // WB
TB = EPW - NWIN * WB
NB = 2
NOUT = NWIN // NB
STRIPE = 624
LSTR = N - 15 * STRIPE


def _seg_body(with_deg, *refs):
    feat_hbm, srcf_hbm, dstf_hbm, zrows_hbm = refs[:4]
    if with_deg:
        parts_hbm, degp_hbm, acc_sh, deg_sh, srcv = refs[4:9]
        k = 9
    else:
        parts_hbm, acc_sh, srcv = refs[4:7]
        k = 7
    dstb = refs[k:k + NB]
    rows = refs[k + NB:k + 2 * NB]
    tdst, trows = refs[k + 2 * NB:k + 2 * NB + 2]
    k += 2
    if with_deg:
        ones_v, deg_v = refs[k + 2 * NB:k + 2 * NB + 2]
        k += 2
    gsem = refs[k + 2 * NB:k + 3 * NB]
    dsem = refs[k + 3 * NB:k + 4 * NB]
    c = lax.axis_index("c")
    s = lax.axis_index("s")
    wid = s * NC + c
    base = pl.multiple_of(wid * EPW, 8)
    st = pl.multiple_of(s * STRIPE, 8)

    pltpu.sync_copy(srcf_hbm.at[pl.ds(base, EPW)], srcv)

    for b in range(NB):
        off = pl.multiple_of(base + b * WB, 8)
        pltpu.async_copy(dstf_hbm.at[pl.ds(off, WB)], dstb[b], dsem[b])
        pltpu.async_copy(feat_hbm.at[srcv.at[pl.ds(b * WB, WB)]], rows[b],
                         gsem[b])

    @pl.when(s < 15)
    def _():
        pltpu.sync_copy(zrows_hbm.at[pl.ds(st, STRIPE)],
                        acc_sh.at[pl.ds(st, STRIPE)])

    @pl.when(s == 15)
    def _():
        pltpu.sync_copy(zrows_hbm.at[pl.ds(15 * STRIPE, LSTR)],
                        acc_sh.at[pl.ds(15 * STRIPE, LSTR)])

    if with_deg:
        def zro(j, carry):
            deg_v[pl.ds(pl.multiple_of(16 * j, 16), 16)] = (
                jnp.zeros((16,), jnp.float32))
            return carry
        lax.fori_loop(0, LSTR // 16, zro, 0)

        @pl.when(s < 15)
        def _():
            pltpu.sync_copy(deg_v.at[pl.ds(0, STRIPE)],
                            deg_sh.at[pl.ds(st, STRIPE)])

        @pl.when(s == 15)
        def _():
            pltpu.sync_copy(deg_v, deg_sh.at[pl.ds(15 * STRIPE, LSTR)])

        for j in range(WB // 16):
            ones_v[pl.ds(16 * j, 16)] = jnp.ones((16,), jnp.float32)
    plsc.subcore_barrier()

    def outer(g, carry):
        for b in range(NB):
            w = g * NB + b
            pltpu.make_async_copy(feat_hbm.at[srcv.at[pl.ds(0, WB)]], rows[b],
                                  gsem[b]).wait()
            pltpu.make_async_copy(dstf_hbm.at[pl.ds(0, WB)], dstb[b],
                                  dsem[b]).wait()
            pltpu.sync_copy(rows[b], acc_sh.at[dstb[b]], add=True)
            if with_deg:
                pltpu.sync_copy(ones_v, deg_sh.at[dstb[b]], add=True)

            @pl.when(g < NOUT - 1)
            def _():
                off = pl.multiple_of(base + (w + NB) * WB, 8)
                pltpu.async_copy(dstf_hbm.at[pl.ds(off, WB)], dstb[b],
                                 dsem[b])
                pltpu.async_copy(
                    feat_hbm.at[srcv.at[pl.ds((w + NB) * WB, WB)]], rows[b],
                    gsem[b])
        return carry

    lax.fori_loop(0, NOUT, outer, 0)

    toff = pl.multiple_of(base + NWIN * WB, 8)
    pltpu.sync_copy(dstf_hbm.at[pl.ds(toff, TB)], tdst)
    pltpu.async_copy(feat_hbm.at[srcv.at[pl.ds(NWIN * WB, TB)]], trows,
                     gsem[0]).wait()
    pltpu.sync_copy(trows, acc_sh.at[tdst], add=True)
    if with_deg:
        pltpu.sync_copy(ones_v.at[pl.ds(0, TB)], deg_sh.at[tdst], add=True)
    plsc.subcore_barrier()

    @pl.when(s < 15)
    def _():
        pltpu.sync_copy(acc_sh.at[pl.ds(st, STRIPE)],
                        parts_hbm.at[c, pl.ds(st, STRIPE)])

    @pl.when(s == 15)
    def _():
        pltpu.sync_copy(acc_sh.at[pl.ds(15 * STRIPE, LSTR)],
                        parts_hbm.at[c, pl.ds(15 * STRIPE, LSTR)])

    if with_deg:
        dgo = pl.multiple_of(c * N, 8)

        @pl.when(s < 15)
        def _():
            pltpu.sync_copy(deg_sh.at[pl.ds(st, STRIPE)],
                            deg_v.at[pl.ds(0, STRIPE)])
            pltpu.sync_copy(deg_v.at[pl.ds(0, STRIPE)],
                            degp_hbm.at[pl.ds(dgo + st, STRIPE)])

        @pl.when(s == 15)
        def _():
            pltpu.sync_copy(deg_sh.at[pl.ds(15 * STRIPE, LSTR)], deg_v)
            pltpu.sync_copy(deg_v,
                            degp_hbm.at[pl.ds(dgo + 15 * STRIPE, LSTR)])


_MESH = plsc.VectorSubcoreMesh(core_axis_name="c", subcore_axis_name="s",
                               num_cores=NC, num_subcores=NS)

_DSTBUFS = [pltpu.VMEM((WB,), jnp.int32) for _ in range(NB)]
_ROWBUFS = [pltpu.VMEM((WB, D), jnp.float32) for _ in range(NB)]
_SEMS = [pltpu.SemaphoreType.DMA for _ in range(2 * NB)]

_seg_deg_call = pl.kernel(
    functools.partial(_seg_body, True),
    out_type=[jax.ShapeDtypeStruct((NC, N, D), jnp.float32),
              jax.ShapeDtypeStruct((NC * N,), jnp.float32)],
    mesh=_MESH,
    scratch_types=[
        pltpu.VMEM_SHARED((N, D), jnp.float32),
        pltpu.VMEM_SHARED((N,), jnp.float32),
        pltpu.VMEM((EPW,), jnp.int32),
        *_DSTBUFS,
        *_ROWBUFS,
        pltpu.VMEM((TB,), jnp.int32),
        pltpu.VMEM((TB, D), jnp.float32),
        pltpu.VMEM((WB,), jnp.float32),
        pltpu.VMEM((LSTR,), jnp.float32),
        *_SEMS,
    ],
)

_seg_call = pl.kernel(
    functools.partial(_seg_body, False),
    out_type=[jax.ShapeDtypeStruct((NC, N, D), jnp.float32)],
    mesh=_MESH,
    scratch_types=[
        pltpu.VMEM_SHARED((N, D), jnp.float32),
        pltpu.VMEM((EPW,), jnp.int32),
        *_DSTBUFS,
        *_ROWBUFS,
        pltpu.VMEM((TB,), jnp.int32),
        pltpu.VMEM((TB, D), jnp.float32),
        *_SEMS,
    ],
)


def _dense_body(relu, nout, bn, parts_ref, degp_ref, feat_ref, ws_ref, wn_ref,
                b_ref, *out_refs):
    p = parts_ref[0] + parts_ref[1]
    d = degp_ref[0, :, 0] + degp_ref[1, :, 0]
    agg = p / jnp.maximum(d, 1.0)[:, None]
    y = (jnp.dot(feat_ref[...], ws_ref[...], preferred_element_type=jnp.float32)
         + jnp.dot(agg, wn_ref[...], preferred_element_type=jnp.float32)
         + b_ref[...])
    if relu:
        y = jnp.maximum(y, 0.0)
    if nout == 1:
        out_refs[0][...] = y
    else:
        h = y.shape[1] // nout
        for i, o in enumerate(out_refs):
            o[...] = y[:, i * h:(i + 1) * h]


def _dense(parts, degp, feat, ws, wn, b, relu, nout):
    bn = 2000
    grid = (N // bn,)
    d_out = ws.shape[1]
    h = d_out // nout
    return pl.pallas_call(
        functools.partial(_dense_body, relu, nout, bn),
        grid=grid,
        in_specs=[
            pl.BlockSpec((NC, bn, D), lambda i: (0, i, 0)),
            pl.BlockSpec((NC, bn, 1), lambda i: (0, i, 0)),
            pl.BlockSpec((bn, D), lambda i: (i, 0)),
            pl.BlockSpec((D, d_out), lambda i: (0, 0)),
            pl.BlockSpec((D, d_out), lambda i: (0, 0)),
            pl.BlockSpec((1, d_out), lambda i: (0, 0)),
        ],
        out_specs=[pl.BlockSpec((bn, h), lambda i: (i, 0))] * nout,
        out_shape=[jax.ShapeDtypeStruct((N, h), jnp.float32)] * nout,
    )(parts, degp, feat, ws, wn, b)


def _esplit_body(ei_ref, src_ref, dst_ref):
    src_ref[...] = ei_ref[0]
    dst_ref[...] = ei_ref[1]


def _esplit(edge_index):
    return pl.pallas_call(
        _esplit_body,
        out_shape=[jax.ShapeDtypeStruct((E,), jnp.int32)] * 2,
    )(edge_index)


def kernel(x, edge_index, W1_self, W1_neigh, b1, Wmu_self, Wmu_neigh, bmu,
           Wls_self, Wls_neigh, bls):
    srcf, dstf = _esplit(edge_index)
    zrows = jnp.zeros((N, D), jnp.float32)

    parts1, degp = _seg_deg_call(x, srcf, dstf, zrows)
    degp = degp.reshape(NC, N, 1)

    (h,) = _dense(parts1, degp, x, W1_self, W1_neigh, b1.reshape(1, D),
                  relu=True, nout=1)

    parts2 = _seg_call(h, srcf, dstf, zrows)[0]

    W2s = jnp.concatenate([Wmu_self, Wls_self], axis=1)
    W2n = jnp.concatenate([Wmu_neigh, Wls_neigh], axis=1)
    b2 = jnp.concatenate([bmu, bls]).reshape(1, D)
    mu, logstd = _dense(parts2, degp, h, W2s, W2n, b2, relu=False, nout=2)
    return mu, logstd


# speedup vs baseline: 1.1020x; 1.1020x over previous
import functools

import jax
import jax.numpy as jnp
from jax import lax
from jax.experimental import pallas as pl
from jax.experimental.pallas import tpu as pltpu
from jax.experimental.pallas import tpu_sc as plsc

N = 10000
D = 128
E = 320000
NC = 2
NS = 16
NW = NC * NS
EPW = E // NW
WB = 64
NWIN = EPW // WB
TB = EPW - NWIN * WB
NB = 4
NOUT = NWIN // NB
STRIPE = 624
LSTR = N - 15 * STRIPE


def _seg_body(with_deg, *refs):
    feat_hbm, srcf_hbm, dstf_hbm, zrows_hbm = refs[:4]
    if with_deg:
        parts_hbm, degp_hbm, acc_sh, deg_sh, srcv = refs[4:9]
        k = 9
    else:
        parts_hbm, acc_sh, srcv = refs[4:7]
        k = 7
    dstb = refs[k:k + NB]
    rows = refs[k + NB:k + 2 * NB]
    tdst, trows = refs[k + 2 * NB:k + 2 * NB + 2]
    k += 2
    if with_deg:
        ones_v, deg_v = refs[k + 2 * NB:k + 2 * NB + 2]
        k += 2
    gsem = refs[k + 2 * NB:k + 3 * NB]
    dsem = refs[k + 3 * NB:k + 4 * NB]
    c = lax.axis_index("c")
    s = lax.axis_index("s")
    wid = s * NC + c
    base = pl.multiple_of(wid * EPW, 8)
    st = pl.multiple_of(s * STRIPE, 8)

    pltpu.sync_copy(srcf_hbm.at[pl.ds(base, EPW)], srcv)

    for b in range(NB):
        off = pl.multiple_of(base + b * WB, 8)
        pltpu.async_copy(dstf_hbm.at[pl.ds(off, WB)], dstb[b], dsem[b])
        pltpu.async_copy(feat_hbm.at[srcv.at[pl.ds(b * WB, WB)]], rows[b],
                         gsem[b])

    @pl.when(s < 15)
    def _():
        pltpu.sync_copy(zrows_hbm.at[pl.ds(st, STRIPE)],
                        acc_sh.at[pl.ds(st, STRIPE)])

    @pl.when(s == 15)
    def _():
        pltpu.sync_copy(zrows_hbm.at[pl.ds(15 * STRIPE, LSTR)],
                        acc_sh.at[pl.ds(15 * STRIPE, LSTR)])

    if with_deg:
        def zro(j, carry):
            deg_v[pl.ds(pl.multiple_of(16 * j, 16), 16)] = (
                jnp.zeros((16,), jnp.float32))
            return carry
        lax.fori_loop(0, LSTR // 16, zro, 0)

        @pl.when(s < 15)
        def _():
            pltpu.sync_copy(deg_v.at[pl.ds(0, STRIPE)],
                            deg_sh.at[pl.ds(st, STRIPE)])

        @pl.when(s == 15)
        def _():
            pltpu.sync_copy(deg_v, deg_sh.at[pl.ds(15 * STRIPE, LSTR)])

        for j in range(WB // 16):
            ones_v[pl.ds(16 * j, 16)] = jnp.ones((16,), jnp.float32)
    plsc.subcore_barrier()

    def outer(g, carry):
        for b in range(NB):
            w = g * NB + b
            pltpu.make_async_copy(feat_hbm.at[srcv.at[pl.ds(0, WB)]], rows[b],
                                  gsem[b]).wait()
            pltpu.make_async_copy(dstf_hbm.at[pl.ds(0, WB)], dstb[b],
                                  dsem[b]).wait()
            pltpu.sync_copy(rows[b], acc_sh.at[dstb[b]], add=True)
            if with_deg:
                pltpu.sync_copy(ones_v, deg_sh.at[dstb[b]], add=True)

            @pl.when(g < NOUT - 1)
            def _():
                off = pl.multiple_of(base + (w + NB) * WB, 8)
                pltpu.async_copy(dstf_hbm.at[pl.ds(off, WB)], dstb[b],
                                 dsem[b])
                pltpu.async_copy(
                    feat_hbm.at[srcv.at[pl.ds((w + NB) * WB, WB)]], rows[b],
                    gsem[b])
        return carry

    lax.fori_loop(0, NOUT, outer, 0)

    toff = pl.multiple_of(base + NWIN * WB, 8)
    pltpu.sync_copy(dstf_hbm.at[pl.ds(toff, TB)], tdst)
    pltpu.async_copy(feat_hbm.at[srcv.at[pl.ds(NWIN * WB, TB)]], trows,
                     gsem[0]).wait()
    pltpu.sync_copy(trows, acc_sh.at[tdst], add=True)
    if with_deg:
        pltpu.sync_copy(ones_v.at[pl.ds(0, TB)], deg_sh.at[tdst], add=True)
    plsc.subcore_barrier()

    @pl.when(s < 15)
    def _():
        pltpu.sync_copy(acc_sh.at[pl.ds(st, STRIPE)],
                        parts_hbm.at[c, pl.ds(st, STRIPE)])

    @pl.when(s == 15)
    def _():
        pltpu.sync_copy(acc_sh.at[pl.ds(15 * STRIPE, LSTR)],
                        parts_hbm.at[c, pl.ds(15 * STRIPE, LSTR)])

    if with_deg:
        dgo = pl.multiple_of(c * N, 8)

        @pl.when(s < 15)
        def _():
            pltpu.sync_copy(deg_sh.at[pl.ds(st, STRIPE)],
                            deg_v.at[pl.ds(0, STRIPE)])
            pltpu.sync_copy(deg_v.at[pl.ds(0, STRIPE)],
                            degp_hbm.at[pl.ds(dgo + st, STRIPE)])

        @pl.when(s == 15)
        def _():
            pltpu.sync_copy(deg_sh.at[pl.ds(15 * STRIPE, LSTR)], deg_v)
            pltpu.sync_copy(deg_v,
                            degp_hbm.at[pl.ds(dgo + 15 * STRIPE, LSTR)])


_MESH = plsc.VectorSubcoreMesh(core_axis_name="c", subcore_axis_name="s",
                               num_cores=NC, num_subcores=NS)

_DSTBUFS = [pltpu.VMEM((WB,), jnp.int32) for _ in range(NB)]
_ROWBUFS = [pltpu.VMEM((WB, D), jnp.float32) for _ in range(NB)]
_SEMS = [pltpu.SemaphoreType.DMA for _ in range(2 * NB)]

_seg_deg_call = pl.kernel(
    functools.partial(_seg_body, True),
    out_type=[jax.ShapeDtypeStruct((NC, N, D), jnp.float32),
              jax.ShapeDtypeStruct((NC * N,), jnp.float32)],
    mesh=_MESH,
    scratch_types=[
        pltpu.VMEM_SHARED((N, D), jnp.float32),
        pltpu.VMEM_SHARED((N,), jnp.float32),
        pltpu.VMEM((EPW,), jnp.int32),
        *_DSTBUFS,
        *_ROWBUFS,
        pltpu.VMEM((TB,), jnp.int32),
        pltpu.VMEM((TB, D), jnp.float32),
        pltpu.VMEM((WB,), jnp.float32),
        pltpu.VMEM((LSTR,), jnp.float32),
        *_SEMS,
    ],
)

_seg_call = pl.kernel(
    functools.partial(_seg_body, False),
    out_type=[jax.ShapeDtypeStruct((NC, N, D), jnp.float32)],
    mesh=_MESH,
    scratch_types=[
        pltpu.VMEM_SHARED((N, D), jnp.float32),
        pltpu.VMEM((EPW,), jnp.int32),
        *_DSTBUFS,
        *_ROWBUFS,
        pltpu.VMEM((TB,), jnp.int32),
        pltpu.VMEM((TB, D), jnp.float32),
        *_SEMS,
    ],
)


def _dense_body(relu, nout, bn, parts_ref, degp_ref, feat_ref, ws_ref, wn_ref,
                b_ref, *out_refs):
    p = parts_ref[0] + parts_ref[1]
    d = degp_ref[0, :, 0] + degp_ref[1, :, 0]
    agg = p / jnp.maximum(d, 1.0)[:, None]
    y = (jnp.dot(feat_ref[...], ws_ref[...], preferred_element_type=jnp.float32)
         + jnp.dot(agg, wn_ref[...], preferred_element_type=jnp.float32)
         + b_ref[...])
    if relu:
        y = jnp.maximum(y, 0.0)
    if nout == 1:
        out_refs[0][...] = y
    else:
        h = y.shape[1] // nout
        for i, o in enumerate(out_refs):
            o[...] = y[:, i * h:(i + 1) * h]


def _dense(parts, degp, feat, ws, wn, b, relu, nout):
    bn = 2000
    grid = (N // bn,)
    d_out = ws.shape[1]
    h = d_out // nout
    return pl.pallas_call(
        functools.partial(_dense_body, relu, nout, bn),
        grid=grid,
        in_specs=[
            pl.BlockSpec((NC, bn, D), lambda i: (0, i, 0)),
            pl.BlockSpec((NC, bn, 1), lambda i: (0, i, 0)),
            pl.BlockSpec((bn, D), lambda i: (i, 0)),
            pl.BlockSpec((D, d_out), lambda i: (0, 0)),
            pl.BlockSpec((D, d_out), lambda i: (0, 0)),
            pl.BlockSpec((1, d_out), lambda i: (0, 0)),
        ],
        out_specs=[pl.BlockSpec((bn, h), lambda i: (i, 0))] * nout,
        out_shape=[jax.ShapeDtypeStruct((N, h), jnp.float32)] * nout,
    )(parts, degp, feat, ws, wn, b)


def _esplit_body(ei_ref, src_ref, dst_ref):
    src_ref[...] = ei_ref[0]
    dst_ref[...] = ei_ref[1]


def _esplit(edge_index):
    return pl.pallas_call(
        _esplit_body,
        out_shape=[jax.ShapeDtypeStruct((E,), jnp.int32)] * 2,
    )(edge_index)


def kernel(x, edge_index, W1_self, W1_neigh, b1, Wmu_self, Wmu_neigh, bmu,
           Wls_self, Wls_neigh, bls):
    srcf, dstf = _esplit(edge_index)
    zrows = jnp.zeros((N, D), jnp.float32)

    parts1, degp = _seg_deg_call(x, srcf, dstf, zrows)
    degp = degp.reshape(NC, N, 1)

    (h,) = _dense(parts1, degp, x, W1_self, W1_neigh, b1.reshape(1, D),
                  relu=True, nout=1)

    parts2 = _seg_call(h, srcf, dstf, zrows)[0]

    W2s = jnp.concatenate([Wmu_self, Wls_self], axis=1)
    W2n = jnp.concatenate([Wmu_neigh, Wls_neigh], axis=1)
    b2 = jnp.concatenate([bmu, bls]).reshape(1, D)
    mu, logstd = _dense(parts2, degp, h, W2s, W2n, b2, relu=False, nout=2)
    return mu, logstd
